# baseline (speedup 1.0000x reference)
import functools

import jax
import jax.numpy as jnp
import numpy as np
from jax.experimental import pallas as pl
from jax.experimental.pallas import tpu as pltpu

_NB = 16


def _dotT(a, x):
    return jax.lax.dot_general(
        a, x, (((0,), (0,)), ((), ())), preferred_element_type=jnp.float32
    )


def _fused_kernel(
    q_ref,
    k_ref,
    v_ref,
    cs_ref,
    cisi_ref,
    r_ref,
    cbig_ref,
    out_ref,
    acc_ref,
    qq_ref,
    *,
    n_nodes,
    nb,
    length,
    n_heads,
    e_dim,
):
    p = pl.program_id(1)
    n = pl.program_id(2)
    fp = 72

    @pl.when(p == 0)
    def _():
        x = q_ref[0, 0]
        for i in range(1, nb):
            x = x + q_ref[0, i]

        @pl.when(n == 0)
        def _():
            acc_ref[...] = x

        @pl.when(n > 0)
        def _():
            acc_ref[...] += x

        @pl.when(n == n_nodes // nb - 1)
        def _():
            qm = acc_ref[...] * (1.0 / n_nodes)
            qqbig = _dotT(cbig_ref[...], qm)
            qq_ref[...] = jnp.concatenate(
                [qqbig[h * 2 * fp : (h + 1) * 2 * fp] for h in range(n_heads)],
                axis=1,
            )

    @pl.when(p == 1)
    def _():
        cs = cs_ref[...]
        cisi = cisi_ref[...]
        r = r_ref[...]
        he = n_heads * e_dim
        qq = qq_ref[...]
        qc = jnp.concatenate([qq[:fp]] * nb, axis=1)
        qs = jnp.concatenate([qq[fp:]] * nb, axis=1)
        k_all = jnp.concatenate([k_ref[0, i] for i in range(nb)], axis=1)
        kk = _dotT(cs, k_all)
        kc = kk[:fp]
        ks = kk[fp:]
        pre = qc * kc + qs * ks
        pim = qc * ks - qs * kc
        p2 = jnp.concatenate([pre, pim], axis=0)
        x_all = jnp.concatenate(
            [
                jnp.dot(
                    p2[:, i * he : (i + 1) * he], r,
                    preferred_element_type=jnp.float32,
                )
                for i in range(nb)
            ],
            axis=1,
        )
        corr_t = _dotT(x_all, cisi)
        m1 = jnp.max(corr_t, axis=1, keepdims=True)
        d_iota = jax.lax.broadcasted_iota(jnp.int32, corr_t.shape, 1)
        i1 = jnp.min(
            jnp.where(corr_t == m1, d_iota, length), axis=1, keepdims=True
        )
        m2 = jnp.max(
            jnp.where(d_iota == i1, -jnp.inf, corr_t), axis=1, keepdims=True
        )
        scale = (jax.nn.sigmoid(m1) + jax.nn.sigmoid(m2)) * 0.5
        for i in range(nb):
            spat = jnp.broadcast_to(
                scale[i * length : i * length + n_heads, :], (n_heads, e_dim)
            )
            v3 = v_ref[0, i].reshape(length, n_heads, e_dim)
            out3 = v3 * spat[None, :, :]
            out_ref[0, i] = out3.reshape(length * n_heads, e_dim)


def kernel(queries, keys, values, attn_mask):
    B, N, L, H, E = queries.shape
    HE = H * E
    LH = L * H
    NB = _NB
    NSTEP = N // NB
    q3 = queries.reshape(B, N, LH, E)
    k4 = keys.reshape(B, N, L, HE)
    v3 = values.reshape(B, N, LH, E)

    F = L // 2 + 1
    FP = 72
    t = np.arange(L)
    f = np.arange(L)
    ang = 2.0 * np.pi * np.outer(t, f) / L
    Cnp = np.cos(ang).astype(np.float32)
    Snp = np.sin(ang).astype(np.float32)
    CSh = np.zeros((L, 2 * FP), dtype=np.float32)
    CSh[:, :F] = Cnp[:, :F]
    CSh[:, FP : FP + F] = Snp[:, :F]
    CS = jnp.asarray(CSh)
    w = np.full((F,), 2.0, dtype=np.float32)
    w[0] = 1.0
    w[L // 2] = 1.0
    CiSih = np.zeros((2 * FP, L), dtype=np.float32)
    CiSih[:F, :] = (w[:, None] * Cnp[:F, :]) * (1.0 / (L * E))
    CiSih[FP : FP + F, :] = (-w[:, None] * Snp[:F, :]) * (1.0 / (L * E))
    CiSi = jnp.asarray(CiSih)
    he = np.arange(HE)
    Rnp = np.zeros((HE, L), dtype=np.float32)
    Rnp[he, he // E] = 1.0
    R = jnp.asarray(Rnp)
    CBIGnp = np.zeros((LH, H * 2 * FP), dtype=np.float32)
    for h in range(H):
        CBIGnp[h::H, h * 2 * FP : (h + 1) * 2 * FP] = CSh
    CBIG = jnp.asarray(CBIGnp)

    def full(shape):
        return pl.BlockSpec(shape, lambda b, p, n: (0,) * len(shape))

    q_spec = pl.BlockSpec(
        (1, NB, LH, E),
        lambda b, p, n: (b, jnp.where(p == 0, n, NSTEP - 1), 0, 0),
    )
    k_spec = pl.BlockSpec(
        (1, NB, L, HE),
        lambda b, p, n: (b, jnp.where(p == 1, n, 0), 0, 0),
    )
    v_spec = pl.BlockSpec(
        (1, NB, LH, E),
        lambda b, p, n: (b, jnp.where(p == 1, n, 0), 0, 0),
    )

    out3 = pl.pallas_call(
        functools.partial(
            _fused_kernel, n_nodes=N, nb=NB, length=L, n_heads=H, e_dim=E
        ),
        grid=(B, 2, NSTEP),
        in_specs=[
            q_spec,
            k_spec,
            v_spec,
            full((L, 2 * FP)),
            full((2 * FP, L)),
            full((HE, L)),
            full((LH, H * 2 * FP)),
        ],
        out_specs=v_spec,
        out_shape=jax.ShapeDtypeStruct((B, N, LH, E), jnp.float32),
        scratch_shapes=[
            pltpu.VMEM((LH, E), jnp.float32),
            pltpu.VMEM((2 * FP, HE), jnp.float32),
        ],
        compiler_params=pltpu.CompilerParams(
            dimension_semantics=("parallel", "arbitrary", "arbitrary"),
            vmem_limit_bytes=64 * 1024 * 1024,
        ),
    )(q3, k4, v3, CS, CiSi, R, CBIG)

    return out3.reshape(B, N, L, H, E)


# speedup vs baseline: 1.1243x; 1.1243x over previous
import functools

import jax
import jax.numpy as jnp
import numpy as np
from jax.experimental import pallas as pl
from jax.experimental.pallas import tpu as pltpu

_NB = 16


def _dotT(a, x):
    return jax.lax.dot_general(
        a, x, (((0,), (0,)), ((), ())), preferred_element_type=jnp.float32
    )


def _fused_kernel(
    q_ref,
    k_ref,
    v_ref,
    cs_ref,
    cisi_ref,
    r_ref,
    out_ref,
    acc_ref,
    qq_ref,
    *,
    n_nodes,
    nb,
    length,
    n_heads,
    e_dim,
):
    p = pl.program_id(1)
    n = pl.program_id(2)
    fp = 72

    @pl.when(p == 0)
    def _():
        x = q_ref[0, 0]
        for i in range(1, nb):
            x = x + q_ref[0, i]

        @pl.when(n == 0)
        def _():
            acc_ref[...] = x

        @pl.when(n > 0)
        def _():
            acc_ref[...] += x

        @pl.when(n == n_nodes // nb - 1)
        def _():
            qm = acc_ref[...] * (1.0 / n_nodes)
            qq_ref[...] = _dotT(cs_ref[...], qm)

    @pl.when(p == 1)
    def _():
        cs = cs_ref[...]
        cisi = cisi_ref[...]
        r = r_ref[...]
        he = n_heads * e_dim
        qq = qq_ref[...]
        qc = jnp.concatenate([qq[:fp]] * nb, axis=1)
        qs = jnp.concatenate([qq[fp:]] * nb, axis=1)
        k_all = jnp.concatenate([k_ref[0, i] for i in range(nb)], axis=1)
        kk = _dotT(cs, k_all)
        kc = kk[:fp]
        ks = kk[fp:]
        pre = qc * kc + qs * ks
        pim = qc * ks - qs * kc
        p2 = jnp.concatenate([pre, pim], axis=0)
        x_all = jnp.concatenate(
            [
                jnp.dot(
                    p2[:, i * he : (i + 1) * he], r,
                    preferred_element_type=jnp.float32,
                )
                for i in range(nb)
            ],
            axis=1,
        )
        corr_t = _dotT(x_all, cisi)
        m1 = jnp.max(corr_t, axis=1, keepdims=True)
        d_iota = jax.lax.broadcasted_iota(jnp.int32, corr_t.shape, 1)
        i1 = jnp.min(
            jnp.where(corr_t == m1, d_iota, length), axis=1, keepdims=True
        )
        m2 = jnp.max(
            jnp.where(d_iota == i1, -jnp.inf, corr_t), axis=1, keepdims=True
        )
        scale = (jax.nn.sigmoid(m1) + jax.nn.sigmoid(m2)) * 0.5
        for i in range(nb):
            spat = jnp.broadcast_to(
                scale[i * length : i * length + n_heads, :], (n_heads, e_dim)
            )
            v3 = v_ref[0, i].reshape(length, n_heads, e_dim)
            out3 = v3 * spat[None, :, :]
            out_ref[0, i] = out3.reshape(length * n_heads, e_dim)


def kernel(queries, keys, values, attn_mask):
    B, N, L, H, E = queries.shape
    HE = H * E
    LH = L * H
    NB = _NB
    NSTEP = N // NB
    q4 = queries.reshape(B, N, L, HE)
    k4 = keys.reshape(B, N, L, HE)
    v3 = values.reshape(B, N, LH, E)

    F = L // 2 + 1
    FP = 72
    t = np.arange(L)
    f = np.arange(L)
    ang = 2.0 * np.pi * np.outer(t, f) / L
    Cnp = np.cos(ang).astype(np.float32)
    Snp = np.sin(ang).astype(np.float32)
    CSh = np.zeros((L, 2 * FP), dtype=np.float32)
    CSh[:, :F] = Cnp[:, :F]
    CSh[:, FP : FP + F] = Snp[:, :F]
    CS = jnp.asarray(CSh)
    w = np.full((F,), 2.0, dtype=np.float32)
    w[0] = 1.0
    w[L // 2] = 1.0
    CiSih = np.zeros((2 * FP, L), dtype=np.float32)
    CiSih[:F, :] = (w[:, None] * Cnp[:F, :]) * (1.0 / (L * E))
    CiSih[FP : FP + F, :] = (-w[:, None] * Snp[:F, :]) * (1.0 / (L * E))
    CiSi = jnp.asarray(CiSih)
    he = np.arange(HE)
    Rnp = np.zeros((HE, L), dtype=np.float32)
    Rnp[he, he // E] = 1.0
    R = jnp.asarray(Rnp)

    def full(shape):
        return pl.BlockSpec(shape, lambda b, p, n: (0,) * len(shape))

    q_spec = pl.BlockSpec(
        (1, NB, L, HE),
        lambda b, p, n: (b, jnp.where(p == 0, n, NSTEP - 1), 0, 0),
    )
    k_spec = pl.BlockSpec(
        (1, NB, L, HE),
        lambda b, p, n: (b, jnp.where(p == 1, n, 0), 0, 0),
    )
    v_spec = pl.BlockSpec(
        (1, NB, LH, E),
        lambda b, p, n: (b, jnp.where(p == 1, n, 0), 0, 0),
    )

    out3 = pl.pallas_call(
        functools.partial(
            _fused_kernel, n_nodes=N, nb=NB, length=L, n_heads=H, e_dim=E
        ),
        grid=(B, 2, NSTEP),
        in_specs=[
            q_spec,
            k_spec,
            v_spec,
            full((L, 2 * FP)),
            full((2 * FP, L)),
            full((HE, L)),
        ],
        out_specs=v_spec,
        out_shape=jax.ShapeDtypeStruct((B, N, LH, E), jnp.float32),
        scratch_shapes=[
            pltpu.VMEM((L, HE), jnp.float32),
            pltpu.VMEM((2 * FP, HE), jnp.float32),
        ],
        compiler_params=pltpu.CompilerParams(
            dimension_semantics=("arbitrary", "arbitrary", "arbitrary")
        ),
    )(q4, k4, v3, CS, CiSi, R)

    return out3.reshape(B, N, L, H, E)
